# baseline (speedup 1.0000x reference)
import jax
import jax.numpy as jnp
from jax import lax
from jax.experimental import pallas as pl
from jax.experimental.pallas import tpu as pltpu


_VMEM_LIMIT = 64 * 1024 * 1024

_TILE_Z = 1024
_DEC_I = 512
_DEC_J = 4096


def _fused(adj, x, w_bf16):
    n = adj.shape[0]
    f = x.shape[1]
    h = w_bf16.shape[1]
    n_z = n // _TILE_Z
    n_i = n // _DEC_I
    n_j = n // _DEC_J
    n_dec = n_i * n_j

    def body(x_ref, w_ref, adj_ref, o_ref, s_ref, z_ref):
        t = pl.program_id(0)

        @pl.when(t == 0)
        def _():
            s_ref[...] = jnp.dot(
                x_ref[...].astype(jnp.bfloat16), w_ref[...],
                preferred_element_type=jnp.float32,
            ).astype(jnp.bfloat16)

        @pl.when(t < n_z)
        def _():
            z_ref[pl.ds(t * _TILE_Z, _TILE_Z), :] = jnp.maximum(
                jnp.dot(
                    adj_ref[...].astype(jnp.bfloat16), s_ref[...],
                    preferred_element_type=jnp.float32,
                ),
                0.0,
            ).astype(jnp.bfloat16)

        @pl.when(t >= n_z)
        def _():
            d = t - n_z
            di = d // n_j
            dj = d % n_j
            zr = z_ref[pl.ds(di * _DEC_I, _DEC_I), :]
            zc = z_ref[pl.ds(dj * _DEC_J, _DEC_J), :]
            logits = lax.dot_general(
                zr, zc,
                dimension_numbers=(((1,), (1,)), ((), ())),
                preferred_element_type=jnp.float32,
            )
            o_ref[...] = 0.5 + 0.5 * jnp.tanh(0.5 * logits)

    def adj_map(t):
        return (jnp.minimum(t, n_z - 1), 0)

    def out_map(t):
        d = jnp.maximum(t - n_z, 0)
        return (d // n_j, d % n_j)

    return pl.pallas_call(
        body,
        out_shape=jax.ShapeDtypeStruct((n, n), jnp.float32),
        grid=(n_z + n_dec,),
        in_specs=[
            pl.BlockSpec((n, f), lambda t: (0, 0)),
            pl.BlockSpec((f, h), lambda t: (0, 0)),
            pl.BlockSpec((_TILE_Z, n), adj_map),
        ],
        out_specs=pl.BlockSpec((_DEC_I, _DEC_J), out_map),
        scratch_shapes=[
            pltpu.VMEM((n, h), jnp.bfloat16),
            pltpu.VMEM((n, h), jnp.bfloat16),
        ],
        compiler_params=pltpu.CompilerParams(
            dimension_semantics=("arbitrary",),
            vmem_limit_bytes=_VMEM_LIMIT,
        ),
    )(x, w_bf16, adj)


def kernel(atac_feature, scrna_feature, adj_norm, edge_attr, gc1_weight):
    del atac_feature, edge_attr

    x = scrna_feature.astype(jnp.float32)
    adj = adj_norm.astype(jnp.float32)
    w_bf16 = gc1_weight.astype(jnp.bfloat16)

    return _fused(adj, x, w_bf16)


# speedup vs baseline: 1.0048x; 1.0048x over previous
import jax
import jax.numpy as jnp
from jax import lax
from jax.experimental import pallas as pl
from jax.experimental.pallas import tpu as pltpu


_VMEM_LIMIT = 64 * 1024 * 1024

_TILE_Z = 1024
_DEC_I = 2048
_DEC_J = 1024


def _fused(adj, x, w_bf16):
    n = adj.shape[0]
    f = x.shape[1]
    h = w_bf16.shape[1]
    n_z = n // _TILE_Z
    n_i = n // _DEC_I
    n_j = n // _DEC_J
    n_dec = n_i * n_j

    def body(x_ref, w_ref, adj_ref, o_ref, s_ref, z_ref):
        t = pl.program_id(0)

        @pl.when(t == 0)
        def _():
            s_ref[...] = jnp.dot(
                x_ref[...].astype(jnp.bfloat16), w_ref[...],
                preferred_element_type=jnp.float32,
            ).astype(jnp.bfloat16)

        @pl.when(t < n_z)
        def _():
            z_ref[pl.ds(t * _TILE_Z, _TILE_Z), :] = jnp.maximum(
                jnp.dot(
                    adj_ref[...].astype(jnp.bfloat16), s_ref[...],
                    preferred_element_type=jnp.float32,
                ),
                0.0,
            ).astype(jnp.bfloat16)

        @pl.when(t >= n_z)
        def _():
            d = t - n_z
            di = d // n_j
            dj = d % n_j
            zr = z_ref[pl.ds(di * _DEC_I, _DEC_I), :]
            zc = z_ref[pl.ds(dj * _DEC_J, _DEC_J), :]
            logits = lax.dot_general(
                zr, zc,
                dimension_numbers=(((1,), (1,)), ((), ())),
                preferred_element_type=jnp.float32,
            )
            o_ref[...] = 0.5 + 0.5 * jnp.tanh(0.5 * logits)

    def adj_map(t):
        return (jnp.minimum(t, n_z - 1), 0)

    def out_map(t):
        d = jnp.maximum(t - n_z, 0)
        return (d // n_j, d % n_j)

    return pl.pallas_call(
        body,
        out_shape=jax.ShapeDtypeStruct((n, n), jnp.float32),
        grid=(n_z + n_dec,),
        in_specs=[
            pl.BlockSpec((n, f), lambda t: (0, 0)),
            pl.BlockSpec((f, h), lambda t: (0, 0)),
            pl.BlockSpec((_TILE_Z, n), adj_map),
        ],
        out_specs=pl.BlockSpec((_DEC_I, _DEC_J), out_map),
        scratch_shapes=[
            pltpu.VMEM((n, h), jnp.bfloat16),
            pltpu.VMEM((n, h), jnp.bfloat16),
        ],
        compiler_params=pltpu.CompilerParams(
            dimension_semantics=("arbitrary",),
            vmem_limit_bytes=_VMEM_LIMIT,
        ),
    )(x, w_bf16, adj)


def kernel(atac_feature, scrna_feature, adj_norm, edge_attr, gc1_weight):
    del atac_feature, edge_attr

    x = scrna_feature.astype(jnp.float32)
    adj = adj_norm.astype(jnp.float32)
    w_bf16 = gc1_weight.astype(jnp.bfloat16)

    return _fused(adj, x, w_bf16)
